# baseline (speedup 1.0000x reference)
import functools
import math

import jax
import jax.numpy as jnp
from jax import lax
from jax.experimental import pallas as pl
from jax.experimental.pallas import tpu as pltpu
from jax.experimental.pallas import tpu_sc as plsc

HIDDEN = 128
NUM_GAUSSIANS = 50
CUTOFF = 10.0
N_NODES = 10000
N_EDGES = 320000
LOG2 = math.log(2.0)

NW = 32
EDGE_BLK = 80
N_CHUNKS = 1
E_PAD = 327680
E_CHUNK = E_PAD // N_CHUNKS
PER_W = E_CHUNK // NW
BLKS_PER_W = PER_W // EDGE_BLK
FILT_BLK = 2560
FILT_BLKS = E_CHUNK // FILT_BLK
ACC_ROWS = 10240
ROWS_PER_SUBCORE = ACC_ROWS // 16


def _ssp(v):
    return jnp.maximum(v, 0.0) + jnp.log(1.0 + jnp.exp(-jnp.abs(v))) - LOG2


def _filter_body(ea_ref, m0t_ref, b0_ref, m2t_ref, b2_ref, out_ref):
    a = ea_ref[...]
    h = jnp.dot(a, m0t_ref[...], preferred_element_type=jnp.float32)
    h = _ssp(h + b0_ref[...])
    w = jnp.dot(h.astype(jnp.bfloat16), m2t_ref[...],
                preferred_element_type=jnp.float32)
    out_ref[...] = w + b2_ref[...]


def _filter_w(ea_chunk, real_blocks, m0t, b0, m2t, b2):
    return pl.pallas_call(
        _filter_body,
        grid=(FILT_BLKS,),
        in_specs=[
            pl.BlockSpec((FILT_BLK, NUM_GAUSSIANS),
                         lambda b: (jnp.minimum(b, real_blocks - 1), 0)),
            pl.BlockSpec((NUM_GAUSSIANS, HIDDEN), lambda b: (0, 0)),
            pl.BlockSpec((1, HIDDEN), lambda b: (0, 0)),
            pl.BlockSpec((HIDDEN, HIDDEN), lambda b: (0, 0)),
            pl.BlockSpec((1, HIDDEN), lambda b: (0, 0)),
        ],
        out_specs=pl.BlockSpec((FILT_BLK, HIDDEN), lambda b: (b, 0)),
        out_shape=jax.ShapeDtypeStruct((E_CHUNK, HIDDEN), jnp.float32),
    )(ea_chunk, m0t, b0, m2t, b2)


def _xh_body(x_ref, wt_ref, out_ref):
    out_ref[...] = jnp.dot(x_ref[...], wt_ref[...],
                           preferred_element_type=jnp.float32)


def _xh(x, lin1_wt):
    blk = 2000
    return pl.pallas_call(
        _xh_body,
        grid=(N_NODES // blk,),
        in_specs=[
            pl.BlockSpec((blk, HIDDEN), lambda b: (b, 0)),
            pl.BlockSpec((HIDDEN, HIDDEN), lambda b: (0, 0)),
        ],
        out_specs=pl.BlockSpec((blk, HIDDEN), lambda b: (b, 0)),
        out_shape=jax.ShapeDtypeStruct((N_NODES, HIDDEN), jnp.float32),
    )(x, lin1_wt)


def _sc_body(chunk, xh, w, cenv, src_h, dst_h, zero, out,
             srcb0, srcb1, srcb2, srcb3, dstb0, dstb1, dstb2, dstb3,
             rows0, rows1, w0, w1, cb0, cb1, acc,
             isem0, isem1, isem2, isem3, dsem0, dsem1):
    c = lax.axis_index("c")
    s = lax.axis_index("s")
    r0 = s * ROWS_PER_SUBCORE
    pltpu.sync_copy(zero.at[pl.ds(r0, ROWS_PER_SUBCORE)],
                    acc.at[pl.ds(r0, ROWS_PER_SUBCORE)])
    plsc.subcore_barrier()

    wid = c * 16 + s
    srcb = (srcb0, srcb1, srcb2, srcb3)
    dstb = (dstb0, dstb1, dstb2, dstb3)
    rows = (rows0, rows1)
    wbuf = (w0, w1)
    cbuf = (cb0, cb1)
    isems = (isem0, isem1, isem2, isem3)
    dsems = (dsem0, dsem1)

    def wbase(i):
        return wid * PER_W + i * EDGE_BLK

    def ebase(i):
        return chunk * E_CHUNK + wbase(i)

    def issue_idx(i, q):
        pltpu.async_copy(src_h.at[pl.ds(ebase(i), EDGE_BLK)], srcb[q], isems[q])
        pltpu.async_copy(dst_h.at[pl.ds(ebase(i), EDGE_BLK)], dstb[q], isems[q])

    def drain_idx(q):
        pltpu.make_async_copy(src_h.at[pl.ds(0, EDGE_BLK)],
                              srcb[q], isems[q]).wait()
        pltpu.make_async_copy(src_h.at[pl.ds(0, EDGE_BLK)],
                              dstb[q], isems[q]).wait()

    def issue_data(i, d, q):
        pltpu.async_copy(xh.at[srcb[q]], rows[d], dsems[d])
        pltpu.async_copy(w.at[pl.ds(wbase(i), EDGE_BLK)], wbuf[d], dsems[d])
        pltpu.async_copy(cenv.at[pl.ds(ebase(i), EDGE_BLK)], cbuf[d], dsems[d])

    def drain_data(d):
        pltpu.make_async_copy(xh.at[pl.ds(0, EDGE_BLK)],
                              rows[d], dsems[d]).wait()
        pltpu.make_async_copy(xh.at[pl.ds(0, EDGE_BLK)],
                              wbuf[d], dsems[d]).wait()
        pltpu.make_async_copy(cenv.at[pl.ds(0, EDGE_BLK)],
                              cbuf[d], dsems[d]).wait()

    for q in range(4):
        issue_idx(q, q)
    drain_idx(0)
    issue_data(0, 0, 0)
    drain_idx(1)
    issue_data(1, 1, 1)

    def blk4(k, carry):
        for m in range(4):
            i = 4 * k + m
            d = m % 2
            q = m
            drain_data(d)

            @plsc.parallel_loop(0, EDGE_BLK, unroll=4)
            def mul(e):
                ce = plsc.load_gather(cbuf[d],
                                      [jnp.full((16,), e, jnp.int32)])
                for j in range(HIDDEN // 16):
                    sl = pl.ds(j * 16, 16)
                    rows[d][e, sl] = rows[d][e, sl] * wbuf[d][e, sl] * ce
            pltpu.sync_copy(rows[d], acc.at[dstb[q]], add=True)

            @pl.when(i + 4 < BLKS_PER_W)
            def _():
                issue_idx(i + 4, q)

            @pl.when(i + 2 < BLKS_PER_W)
            def _():
                drain_idx((m + 2) % 4)
                issue_data(i + 2, d, (m + 2) % 4)
        return carry

    lax.fori_loop(0, BLKS_PER_W // 4, blk4, 0)
    plsc.subcore_barrier()
    pltpu.sync_copy(acc.at[pl.ds(r0, ROWS_PER_SUBCORE)],
                    out.at[c, pl.ds(r0, ROWS_PER_SUBCORE)])


def _sc_scatter(chunk, xh, w, cenv, src_h, dst_h, zero):
    mesh = plsc.VectorSubcoreMesh(core_axis_name="c", subcore_axis_name="s")
    kfn = functools.partial(
        pl.kernel,
        mesh=mesh,
        compiler_params=pltpu.CompilerParams(needs_layout_passes=False),
        out_type=jax.ShapeDtypeStruct((2, ACC_ROWS, HIDDEN), jnp.float32),
        scratch_types=[
            pltpu.VMEM((EDGE_BLK,), jnp.int32),
            pltpu.VMEM((EDGE_BLK,), jnp.int32),
            pltpu.VMEM((EDGE_BLK,), jnp.int32),
            pltpu.VMEM((EDGE_BLK,), jnp.int32),
            pltpu.VMEM((EDGE_BLK,), jnp.int32),
            pltpu.VMEM((EDGE_BLK,), jnp.int32),
            pltpu.VMEM((EDGE_BLK,), jnp.int32),
            pltpu.VMEM((EDGE_BLK,), jnp.int32),
            pltpu.VMEM((EDGE_BLK, HIDDEN), jnp.float32),
            pltpu.VMEM((EDGE_BLK, HIDDEN), jnp.float32),
            pltpu.VMEM((EDGE_BLK, HIDDEN), jnp.float32),
            pltpu.VMEM((EDGE_BLK, HIDDEN), jnp.float32),
            pltpu.VMEM((EDGE_BLK,), jnp.float32),
            pltpu.VMEM((EDGE_BLK,), jnp.float32),
            pltpu.VMEM_SHARED((ACC_ROWS, HIDDEN), jnp.float32),
            pltpu.SemaphoreType.DMA,
            pltpu.SemaphoreType.DMA,
            pltpu.SemaphoreType.DMA,
            pltpu.SemaphoreType.DMA,
            pltpu.SemaphoreType.DMA,
            pltpu.SemaphoreType.DMA,
        ],
    )(functools.partial(_sc_body, chunk))
    return kfn(xh, w, cenv, src_h, dst_h, zero)


def _tail_body(p0_ref, l2t_ref, b2_ref, lwt_ref, lb_ref, out_ref):
    s = p0_ref[0] + p0_ref[1]
    t = _ssp(jnp.dot(s, l2t_ref[...], preferred_element_type=jnp.float32)
             + b2_ref[...])
    out_ref[...] = jnp.dot(t, lwt_ref[...],
                           preferred_element_type=jnp.float32) + lb_ref[...]


def _tail(parts0, lin2_wt, lin2_b, lin_wt, lin_b):
    blk = 2000
    return pl.pallas_call(
        _tail_body,
        grid=(N_NODES // blk,),
        in_specs=[
            pl.BlockSpec((2, blk, HIDDEN), lambda b: (0, b, 0)),
            pl.BlockSpec((HIDDEN, HIDDEN), lambda b: (0, 0)),
            pl.BlockSpec((1, HIDDEN), lambda b: (0, 0)),
            pl.BlockSpec((HIDDEN, HIDDEN), lambda b: (0, 0)),
            pl.BlockSpec((1, HIDDEN), lambda b: (0, 0)),
        ],
        out_specs=pl.BlockSpec((blk, HIDDEN), lambda b: (b, 0)),
        out_shape=jax.ShapeDtypeStruct((N_NODES, HIDDEN), jnp.float32),
    )(parts0, lin2_wt, lin2_b, lin_wt, lin_b)


def kernel(x, edge_index, edge_weight, edge_attr,
           mlp0_w, mlp0_b, mlp2_w, mlp2_b,
           lin1_w, lin2_w, lin2_b, lin_w, lin_b):
    pad = E_PAD - N_EDGES
    src = jnp.concatenate([edge_index[0].astype(jnp.int32),
                           jnp.zeros((pad,), jnp.int32)])
    dst = jnp.concatenate([edge_index[1].astype(jnp.int32),
                           jnp.zeros((pad,), jnp.int32)])
    cenv = jnp.concatenate(
        [0.5 * (jnp.cos(edge_weight.astype(jnp.float32) * (math.pi / CUTOFF))
                + 1.0),
         jnp.zeros((pad,), jnp.float32)])
    m0t = mlp0_w.T.astype(jnp.bfloat16)
    m2t = mlp2_w.T.astype(jnp.bfloat16)
    b0 = mlp0_b.reshape(1, HIDDEN)
    b2 = mlp2_b.reshape(1, HIDDEN)
    xh = _xh(x, lin1_w.T)
    zero = jnp.zeros((ACC_ROWS, HIDDEN), jnp.float32)
    ea = edge_attr.astype(jnp.bfloat16)
    w_e = _filter_w(ea, N_EDGES // FILT_BLK, m0t, b0, m2t, b2)
    parts = _sc_scatter(0, xh, w_e, cenv, src, dst, zero)
    return _tail(parts, lin2_w.T, lin2_b.reshape(1, HIDDEN),
                 lin_w.T, lin_b.reshape(1, HIDDEN))


# speedup vs baseline: 1.0394x; 1.0394x over previous
import functools
import math

import jax
import jax.numpy as jnp
from jax import lax
from jax.experimental import pallas as pl
from jax.experimental.pallas import tpu as pltpu
from jax.experimental.pallas import tpu_sc as plsc

HIDDEN = 128
NUM_GAUSSIANS = 50
CUTOFF = 10.0
N_NODES = 10000
N_EDGES = 320000
LOG2 = math.log(2.0)

NW = 32
EDGE_BLK = 80
E_PAD = 327680
HALF_E = E_PAD // 2
PER_W = E_PAD // NW
BLKS_PER_W = PER_W // EDGE_BLK
FILT_BLK = 2560
FILT_BLKS = HALF_E // FILT_BLK
ACC_ROWS = 10240
ROWS_PER_SUBCORE = ACC_ROWS // 16


def _ssp(v):
    return jnp.maximum(v, 0.0) + jnp.log(1.0 + jnp.exp(-jnp.abs(v))) - LOG2


def _filter_body(ea_ref, m0t_ref, b0_ref, m2t_ref, b2_ref, out_ref):
    a = ea_ref[...]
    h = jnp.dot(a, m0t_ref[...], preferred_element_type=jnp.float32)
    h = _ssp(h + b0_ref[...])
    w = jnp.dot(h.astype(jnp.bfloat16), m2t_ref[...],
                preferred_element_type=jnp.float32)
    out_ref[...] = w + b2_ref[...]


def _filter_w(ea_chunk, real_blocks, m0t, b0, m2t, b2):
    return pl.pallas_call(
        _filter_body,
        grid=(FILT_BLKS,),
        in_specs=[
            pl.BlockSpec((FILT_BLK, NUM_GAUSSIANS),
                         lambda b: (jnp.minimum(b, real_blocks - 1), 0)),
            pl.BlockSpec((NUM_GAUSSIANS, HIDDEN), lambda b: (0, 0)),
            pl.BlockSpec((1, HIDDEN), lambda b: (0, 0)),
            pl.BlockSpec((HIDDEN, HIDDEN), lambda b: (0, 0)),
            pl.BlockSpec((1, HIDDEN), lambda b: (0, 0)),
        ],
        out_specs=pl.BlockSpec((FILT_BLK, HIDDEN), lambda b: (b, 0)),
        out_shape=jax.ShapeDtypeStruct((HALF_E, HIDDEN), jnp.float32),
    )(ea_chunk, m0t, b0, m2t, b2)


def _xh_body(x_ref, wt_ref, out_ref):
    out_ref[...] = jnp.dot(x_ref[...], wt_ref[...],
                           preferred_element_type=jnp.float32)


def _xh(x, lin1_wt):
    blk = 2000
    return pl.pallas_call(
        _xh_body,
        grid=(N_NODES // blk,),
        in_specs=[
            pl.BlockSpec((blk, HIDDEN), lambda b: (b, 0)),
            pl.BlockSpec((HIDDEN, HIDDEN), lambda b: (0, 0)),
        ],
        out_specs=pl.BlockSpec((blk, HIDDEN), lambda b: (b, 0)),
        out_shape=jax.ShapeDtypeStruct((N_NODES, HIDDEN), jnp.float32),
    )(x, lin1_wt)


def _sc_body(xh, wc0, wc1, cenv, src_h, dst_h, zero, out,
             srcb0, srcb1, srcb2, srcb3, dstb0, dstb1, dstb2, dstb3,
             rows0, rows1, w0, w1, cb0, cb1, acc,
             isem0, isem1, isem2, isem3, dsem0, dsem1):
    c = lax.axis_index("c")
    s = lax.axis_index("s")
    r0 = s * ROWS_PER_SUBCORE
    pltpu.sync_copy(zero.at[pl.ds(r0, ROWS_PER_SUBCORE)],
                    acc.at[pl.ds(r0, ROWS_PER_SUBCORE)])
    plsc.subcore_barrier()

    wid = c * 16 + s
    srcb = (srcb0, srcb1, srcb2, srcb3)
    dstb = (dstb0, dstb1, dstb2, dstb3)
    rows = (rows0, rows1)
    wbuf = (w0, w1)
    cbuf = (cb0, cb1)
    isems = (isem0, isem1, isem2, isem3)
    dsems = (dsem0, dsem1)

    def wbase(i):
        return s * PER_W + i * EDGE_BLK

    def ebase(i):
        return wid * PER_W + i * EDGE_BLK

    def issue_idx(i, q):
        pltpu.async_copy(src_h.at[pl.ds(ebase(i), EDGE_BLK)], srcb[q], isems[q])
        pltpu.async_copy(dst_h.at[pl.ds(ebase(i), EDGE_BLK)], dstb[q], isems[q])

    def drain_idx(q):
        pltpu.make_async_copy(src_h.at[pl.ds(0, EDGE_BLK)],
                              srcb[q], isems[q]).wait()
        pltpu.make_async_copy(src_h.at[pl.ds(0, EDGE_BLK)],
                              dstb[q], isems[q]).wait()

    def issue_data(i, d, q):
        pltpu.async_copy(xh.at[srcb[q]], rows[d], dsems[d])

        @pl.when(c == 0)
        def _():
            pltpu.async_copy(wc0.at[pl.ds(wbase(i), EDGE_BLK)],
                             wbuf[d], dsems[d])

        @pl.when(c == 1)
        def _():
            pltpu.async_copy(wc1.at[pl.ds(wbase(i), EDGE_BLK)],
                             wbuf[d], dsems[d])

        pltpu.async_copy(cenv.at[pl.ds(ebase(i), EDGE_BLK)], cbuf[d], dsems[d])

    def drain_data(d):
        pltpu.make_async_copy(xh.at[pl.ds(0, EDGE_BLK)],
                              rows[d], dsems[d]).wait()
        pltpu.make_async_copy(xh.at[pl.ds(0, EDGE_BLK)],
                              wbuf[d], dsems[d]).wait()
        pltpu.make_async_copy(cenv.at[pl.ds(0, EDGE_BLK)],
                              cbuf[d], dsems[d]).wait()

    for q in range(4):
        issue_idx(q, q)
    drain_idx(0)
    issue_data(0, 0, 0)
    drain_idx(1)
    issue_data(1, 1, 1)

    def blk4(k, carry):
        for m in range(4):
            i = 4 * k + m
            d = m % 2
            q = m
            drain_data(d)

            @plsc.parallel_loop(0, EDGE_BLK, unroll=2)
            def mul(e):
                ce = plsc.load_gather(cbuf[d],
                                      [jnp.full((16,), e, jnp.int32)])
                for j in range(HIDDEN // 16):
                    sl = pl.ds(j * 16, 16)
                    rows[d][e, sl] = rows[d][e, sl] * wbuf[d][e, sl] * ce
            pltpu.sync_copy(rows[d], acc.at[dstb[q]], add=True)

            @pl.when(i + 4 < BLKS_PER_W)
            def _():
                issue_idx(i + 4, q)

            @pl.when(i + 2 < BLKS_PER_W)
            def _():
                drain_idx((m + 2) % 4)
                issue_data(i + 2, d, (m + 2) % 4)
        return carry

    lax.fori_loop(0, BLKS_PER_W // 4, blk4, 0)
    plsc.subcore_barrier()
    pltpu.sync_copy(acc.at[pl.ds(r0, ROWS_PER_SUBCORE)],
                    out.at[c, pl.ds(r0, ROWS_PER_SUBCORE)])


def _sc_scatter(xh, wc0, wc1, cenv, src_h, dst_h, zero):
    mesh = plsc.VectorSubcoreMesh(core_axis_name="c", subcore_axis_name="s")
    kfn = functools.partial(
        pl.kernel,
        mesh=mesh,
        compiler_params=pltpu.CompilerParams(needs_layout_passes=False),
        out_type=jax.ShapeDtypeStruct((2, ACC_ROWS, HIDDEN), jnp.float32),
        scratch_types=[
            pltpu.VMEM((EDGE_BLK,), jnp.int32),
            pltpu.VMEM((EDGE_BLK,), jnp.int32),
            pltpu.VMEM((EDGE_BLK,), jnp.int32),
            pltpu.VMEM((EDGE_BLK,), jnp.int32),
            pltpu.VMEM((EDGE_BLK,), jnp.int32),
            pltpu.VMEM((EDGE_BLK,), jnp.int32),
            pltpu.VMEM((EDGE_BLK,), jnp.int32),
            pltpu.VMEM((EDGE_BLK,), jnp.int32),
            pltpu.VMEM((EDGE_BLK, HIDDEN), jnp.float32),
            pltpu.VMEM((EDGE_BLK, HIDDEN), jnp.float32),
            pltpu.VMEM((EDGE_BLK, HIDDEN), jnp.float32),
            pltpu.VMEM((EDGE_BLK, HIDDEN), jnp.float32),
            pltpu.VMEM((EDGE_BLK,), jnp.float32),
            pltpu.VMEM((EDGE_BLK,), jnp.float32),
            pltpu.VMEM_SHARED((ACC_ROWS, HIDDEN), jnp.float32),
            pltpu.SemaphoreType.DMA,
            pltpu.SemaphoreType.DMA,
            pltpu.SemaphoreType.DMA,
            pltpu.SemaphoreType.DMA,
            pltpu.SemaphoreType.DMA,
            pltpu.SemaphoreType.DMA,
        ],
    )(_sc_body)
    return kfn(xh, wc0, wc1, cenv, src_h, dst_h, zero)


def _tail_body(p0_ref, l2t_ref, b2_ref, lwt_ref, lb_ref, out_ref):
    s = p0_ref[0] + p0_ref[1]
    t = _ssp(jnp.dot(s, l2t_ref[...], preferred_element_type=jnp.float32)
             + b2_ref[...])
    out_ref[...] = jnp.dot(t, lwt_ref[...],
                           preferred_element_type=jnp.float32) + lb_ref[...]


def _tail(parts0, lin2_wt, lin2_b, lin_wt, lin_b):
    blk = 2000
    return pl.pallas_call(
        _tail_body,
        grid=(N_NODES // blk,),
        in_specs=[
            pl.BlockSpec((2, blk, HIDDEN), lambda b: (0, b, 0)),
            pl.BlockSpec((HIDDEN, HIDDEN), lambda b: (0, 0)),
            pl.BlockSpec((1, HIDDEN), lambda b: (0, 0)),
            pl.BlockSpec((HIDDEN, HIDDEN), lambda b: (0, 0)),
            pl.BlockSpec((1, HIDDEN), lambda b: (0, 0)),
        ],
        out_specs=pl.BlockSpec((blk, HIDDEN), lambda b: (b, 0)),
        out_shape=jax.ShapeDtypeStruct((N_NODES, HIDDEN), jnp.float32),
    )(parts0, lin2_wt, lin2_b, lin_wt, lin_b)


def kernel(x, edge_index, edge_weight, edge_attr,
           mlp0_w, mlp0_b, mlp2_w, mlp2_b,
           lin1_w, lin2_w, lin2_b, lin_w, lin_b):
    pad = E_PAD - N_EDGES
    src = jnp.concatenate([edge_index[0].astype(jnp.int32),
                           jnp.zeros((pad,), jnp.int32)])
    dst = jnp.concatenate([edge_index[1].astype(jnp.int32),
                           jnp.zeros((pad,), jnp.int32)])
    cenv = jnp.concatenate(
        [0.5 * (jnp.cos(edge_weight.astype(jnp.float32) * (math.pi / CUTOFF))
                + 1.0),
         jnp.zeros((pad,), jnp.float32)])
    m0t = mlp0_w.T.astype(jnp.bfloat16)
    m2t = mlp2_w.T.astype(jnp.bfloat16)
    b0 = mlp0_b.reshape(1, HIDDEN)
    b2 = mlp2_b.reshape(1, HIDDEN)
    xh = _xh(x, lin1_w.T)
    zero = jnp.zeros((ACC_ROWS, HIDDEN), jnp.float32)
    ea0 = edge_attr[:HALF_E].astype(jnp.bfloat16)
    ea1 = edge_attr[HALF_E:N_EDGES].astype(jnp.bfloat16)
    wc0 = _filter_w(ea0, FILT_BLKS, m0t, b0, m2t, b2)
    wc1 = _filter_w(ea1, (N_EDGES - HALF_E) // FILT_BLK, m0t, b0, m2t, b2)
    parts = _sc_scatter(xh, wc0, wc1, cenv, src, dst, zero)
    return _tail(parts, lin2_w.T, lin2_b.reshape(1, HIDDEN),
                 lin_w.T, lin_b.reshape(1, HIDDEN))


# speedup vs baseline: 1.0509x; 1.0110x over previous
import functools
import math

import jax
import jax.numpy as jnp
from jax import lax
from jax.experimental import pallas as pl
from jax.experimental.pallas import tpu as pltpu
from jax.experimental.pallas import tpu_sc as plsc

HIDDEN = 128
NUM_GAUSSIANS = 50
CUTOFF = 10.0
N_NODES = 10000
N_EDGES = 320000
LOG2 = math.log(2.0)

NW = 32
EDGE_BLK = 80
E_PAD = 327680
HALF_E = E_PAD // 2
PER_W = E_PAD // NW
BLKS_PER_W = PER_W // EDGE_BLK
FILT_BLK = 2560
FILT_BLKS = HALF_E // FILT_BLK
ACC_ROWS = 10240
ROWS_PER_SUBCORE = ACC_ROWS // 16


def _ssp(v):
    return jnp.maximum(v, 0.0) + jnp.log(1.0 + jnp.exp(-jnp.abs(v))) - LOG2


def _filter_body(ea_ref, m0t_ref, b0_ref, m2t_ref, b2_ref, out_ref):
    a = ea_ref[...]
    h = jnp.dot(a, m0t_ref[...], preferred_element_type=jnp.float32)
    h = _ssp(h + b0_ref[...])
    w = jnp.dot(h.astype(jnp.bfloat16), m2t_ref[...],
                preferred_element_type=jnp.float32)
    out_ref[...] = w + b2_ref[...]


def _filter_w(ea, off, m0t, b0, m2t, b2):
    last = N_EDGES // FILT_BLK - 1
    return pl.pallas_call(
        _filter_body,
        grid=(FILT_BLKS,),
        in_specs=[
            pl.BlockSpec((FILT_BLK, NUM_GAUSSIANS),
                         lambda b: (jnp.minimum(b + off, last), 0)),
            pl.BlockSpec((NUM_GAUSSIANS, HIDDEN), lambda b: (0, 0)),
            pl.BlockSpec((1, HIDDEN), lambda b: (0, 0)),
            pl.BlockSpec((HIDDEN, HIDDEN), lambda b: (0, 0)),
            pl.BlockSpec((1, HIDDEN), lambda b: (0, 0)),
        ],
        out_specs=pl.BlockSpec((FILT_BLK, HIDDEN), lambda b: (b, 0)),
        out_shape=jax.ShapeDtypeStruct((HALF_E, HIDDEN), jnp.float32),
    )(ea, m0t, b0, m2t, b2)


def _xh_body(x_ref, wt_ref, out_ref):
    out_ref[...] = jnp.dot(x_ref[...], wt_ref[...],
                           preferred_element_type=jnp.float32)


def _xh(x, lin1_wt):
    blk = 2000
    return pl.pallas_call(
        _xh_body,
        grid=(N_NODES // blk,),
        in_specs=[
            pl.BlockSpec((blk, HIDDEN), lambda b: (b, 0)),
            pl.BlockSpec((HIDDEN, HIDDEN), lambda b: (0, 0)),
        ],
        out_specs=pl.BlockSpec((blk, HIDDEN), lambda b: (b, 0)),
        out_shape=jax.ShapeDtypeStruct((N_NODES, HIDDEN), jnp.float32),
    )(x, lin1_wt)


def _sc_body(xh, wc0, wc1, cenv, src_h, dst_h, zero, out,
             srcb0, srcb1, srcb2, srcb3, dstb0, dstb1, dstb2, dstb3,
             rows0, rows1, w0, w1, cb0, cb1, acc,
             isem0, isem1, isem2, isem3, dsem0, dsem1):
    c = lax.axis_index("c")
    s = lax.axis_index("s")
    r0 = s * ROWS_PER_SUBCORE
    pltpu.sync_copy(zero.at[pl.ds(r0, ROWS_PER_SUBCORE)],
                    acc.at[pl.ds(r0, ROWS_PER_SUBCORE)])
    plsc.subcore_barrier()

    wid = c * 16 + s
    srcb = (srcb0, srcb1, srcb2, srcb3)
    dstb = (dstb0, dstb1, dstb2, dstb3)
    rows = (rows0, rows1)
    wbuf = (w0, w1)
    cbuf = (cb0, cb1)
    isems = (isem0, isem1, isem2, isem3)
    dsems = (dsem0, dsem1)

    def wbase(i):
        return s * PER_W + i * EDGE_BLK

    def ebase(i):
        return wid * PER_W + i * EDGE_BLK

    def issue_idx(i, q):
        pltpu.async_copy(src_h.at[pl.ds(ebase(i), EDGE_BLK)], srcb[q], isems[q])
        pltpu.async_copy(dst_h.at[pl.ds(ebase(i), EDGE_BLK)], dstb[q], isems[q])

    def drain_idx(q):
        pltpu.make_async_copy(src_h.at[pl.ds(0, EDGE_BLK)],
                              srcb[q], isems[q]).wait()
        pltpu.make_async_copy(src_h.at[pl.ds(0, EDGE_BLK)],
                              dstb[q], isems[q]).wait()

    def issue_data(i, d, q):
        pltpu.async_copy(xh.at[srcb[q]], rows[d], dsems[d])

        @pl.when(c == 0)
        def _():
            pltpu.async_copy(wc0.at[pl.ds(wbase(i), EDGE_BLK)],
                             wbuf[d], dsems[d])

        @pl.when(c == 1)
        def _():
            pltpu.async_copy(wc1.at[pl.ds(wbase(i), EDGE_BLK)],
                             wbuf[d], dsems[d])

        pltpu.async_copy(cenv.at[pl.ds(ebase(i), EDGE_BLK)], cbuf[d], dsems[d])

    def drain_data(d):
        pltpu.make_async_copy(xh.at[pl.ds(0, EDGE_BLK)],
                              rows[d], dsems[d]).wait()
        pltpu.make_async_copy(xh.at[pl.ds(0, EDGE_BLK)],
                              wbuf[d], dsems[d]).wait()
        pltpu.make_async_copy(cenv.at[pl.ds(0, EDGE_BLK)],
                              cbuf[d], dsems[d]).wait()

    for q in range(4):
        issue_idx(q, q)
    drain_idx(0)
    issue_data(0, 0, 0)
    drain_idx(1)
    issue_data(1, 1, 1)

    def blk4(k, carry):
        for m in range(4):
            i = 4 * k + m
            d = m % 2
            q = m
            drain_data(d)

            @plsc.parallel_loop(0, EDGE_BLK, unroll=2)
            def mul(e):
                ce = plsc.load_gather(cbuf[d],
                                      [jnp.full((16,), e, jnp.int32)])
                for j in range(HIDDEN // 16):
                    sl = pl.ds(j * 16, 16)
                    rows[d][e, sl] = rows[d][e, sl] * wbuf[d][e, sl] * ce
            pltpu.sync_copy(rows[d], acc.at[dstb[q]], add=True)

            @pl.when(i + 4 < BLKS_PER_W)
            def _():
                issue_idx(i + 4, q)

            @pl.when(i + 2 < BLKS_PER_W)
            def _():
                drain_idx((m + 2) % 4)
                issue_data(i + 2, d, (m + 2) % 4)
        return carry

    lax.fori_loop(0, BLKS_PER_W // 4, blk4, 0)
    plsc.subcore_barrier()
    pltpu.sync_copy(acc.at[pl.ds(r0, ROWS_PER_SUBCORE)],
                    out.at[c, pl.ds(r0, ROWS_PER_SUBCORE)])


def _sc_scatter(xh, wc0, wc1, cenv, src_h, dst_h, zero):
    mesh = plsc.VectorSubcoreMesh(core_axis_name="c", subcore_axis_name="s")
    kfn = functools.partial(
        pl.kernel,
        mesh=mesh,
        compiler_params=pltpu.CompilerParams(needs_layout_passes=False),
        out_type=jax.ShapeDtypeStruct((2, ACC_ROWS, HIDDEN), jnp.float32),
        scratch_types=[
            pltpu.VMEM((EDGE_BLK,), jnp.int32),
            pltpu.VMEM((EDGE_BLK,), jnp.int32),
            pltpu.VMEM((EDGE_BLK,), jnp.int32),
            pltpu.VMEM((EDGE_BLK,), jnp.int32),
            pltpu.VMEM((EDGE_BLK,), jnp.int32),
            pltpu.VMEM((EDGE_BLK,), jnp.int32),
            pltpu.VMEM((EDGE_BLK,), jnp.int32),
            pltpu.VMEM((EDGE_BLK,), jnp.int32),
            pltpu.VMEM((EDGE_BLK, HIDDEN), jnp.float32),
            pltpu.VMEM((EDGE_BLK, HIDDEN), jnp.float32),
            pltpu.VMEM((EDGE_BLK, HIDDEN), jnp.float32),
            pltpu.VMEM((EDGE_BLK, HIDDEN), jnp.float32),
            pltpu.VMEM((EDGE_BLK,), jnp.float32),
            pltpu.VMEM((EDGE_BLK,), jnp.float32),
            pltpu.VMEM_SHARED((ACC_ROWS, HIDDEN), jnp.float32),
            pltpu.SemaphoreType.DMA,
            pltpu.SemaphoreType.DMA,
            pltpu.SemaphoreType.DMA,
            pltpu.SemaphoreType.DMA,
            pltpu.SemaphoreType.DMA,
            pltpu.SemaphoreType.DMA,
        ],
    )(_sc_body)
    return kfn(xh, wc0, wc1, cenv, src_h, dst_h, zero)


def _tail_body(p0_ref, l2t_ref, b2_ref, lwt_ref, lb_ref, out_ref):
    s = p0_ref[0] + p0_ref[1]
    t = _ssp(jnp.dot(s, l2t_ref[...], preferred_element_type=jnp.float32)
             + b2_ref[...])
    out_ref[...] = jnp.dot(t, lwt_ref[...],
                           preferred_element_type=jnp.float32) + lb_ref[...]


def _tail(parts0, lin2_wt, lin2_b, lin_wt, lin_b):
    blk = 2000
    return pl.pallas_call(
        _tail_body,
        grid=(N_NODES // blk,),
        in_specs=[
            pl.BlockSpec((2, blk, HIDDEN), lambda b: (0, b, 0)),
            pl.BlockSpec((HIDDEN, HIDDEN), lambda b: (0, 0)),
            pl.BlockSpec((1, HIDDEN), lambda b: (0, 0)),
            pl.BlockSpec((HIDDEN, HIDDEN), lambda b: (0, 0)),
            pl.BlockSpec((1, HIDDEN), lambda b: (0, 0)),
        ],
        out_specs=pl.BlockSpec((blk, HIDDEN), lambda b: (b, 0)),
        out_shape=jax.ShapeDtypeStruct((N_NODES, HIDDEN), jnp.float32),
    )(parts0, lin2_wt, lin2_b, lin_wt, lin_b)


def kernel(x, edge_index, edge_weight, edge_attr,
           mlp0_w, mlp0_b, mlp2_w, mlp2_b,
           lin1_w, lin2_w, lin2_b, lin_w, lin_b):
    pad = E_PAD - N_EDGES
    src = jnp.concatenate([edge_index[0].astype(jnp.int32),
                           jnp.zeros((pad,), jnp.int32)])
    dst = jnp.concatenate([edge_index[1].astype(jnp.int32),
                           jnp.zeros((pad,), jnp.int32)])
    cenv = jnp.concatenate(
        [0.5 * (jnp.cos(edge_weight.astype(jnp.float32) * (math.pi / CUTOFF))
                + 1.0),
         jnp.zeros((pad,), jnp.float32)])
    m0t = mlp0_w.T.astype(jnp.bfloat16)
    m2t = mlp2_w.T.astype(jnp.bfloat16)
    b0 = mlp0_b.reshape(1, HIDDEN)
    b2 = mlp2_b.reshape(1, HIDDEN)
    xh = _xh(x, lin1_w.T)
    zero = jnp.zeros((ACC_ROWS, HIDDEN), jnp.float32)
    ea = edge_attr.astype(jnp.bfloat16)
    wc0 = _filter_w(ea, 0, m0t, b0, m2t, b2)
    wc1 = _filter_w(ea, FILT_BLKS, m0t, b0, m2t, b2)
    parts = _sc_scatter(xh, wc0, wc1, cenv, src, dst, zero)
    return _tail(parts, lin2_w.T, lin2_b.reshape(1, HIDDEN),
                 lin_w.T, lin_b.reshape(1, HIDDEN))
